# baseline (speedup 1.0000x reference)
import functools

import jax
import jax.numpy as jnp
from jax import lax
from jax.experimental import pallas as pl
from jax.experimental.pallas import tpu as pltpu
from jax.experimental.pallas import tpu_sc as plsc

N = 10000
E = 320000
D = 128
OUT = 2
NC = 2
NS = 16
NW = NC * NS
L = 16
NP = 10240
EPW = E // NW

EBIG = 79 * 128
ESM = 78 * 128
NBIG = (E - NW * ESM) // (EBIG - ESM)

_SC_MESH = dict(core_axis_name="c", subcore_axis_name="s")


def _mesh():
    return plsc.VectorSubcoreMesh(num_cores=NC, num_subcores=NS, **_SC_MESH)


def _sc_params():
    return pltpu.CompilerParams(needs_layout_passes=False)


def _zero_vmem(zc_hbm, refs):
    for r in refs:
        pltpu.sync_copy(zc_hbm, r)


def _sc_degree(edge_index, zc):
    @functools.partial(
        pl.kernel,
        out_type=(jax.ShapeDtypeStruct((E,), jnp.int32),
                  jax.ShapeDtypeStruct((E,), jnp.int32),
                  jax.ShapeDtypeStruct((NW, NP), jnp.float32)),
        mesh=_mesh(),
        compiler_params=_sc_params(),
        scratch_types=[
            pltpu.VMEM((2, EBIG), jnp.int32),
            pltpu.VMEM((NP,), jnp.float32),
            pltpu.VMEM((NP,), jnp.float32),
            pltpu.VMEM((NP,), jnp.float32),
            pltpu.VMEM((NP,), jnp.float32),
        ],
    )
    def k(edge_hbm, zc_hbm, src_out, dst_out, degp_out, ed_v, a0, a1, a2, a3):
        wid = lax.axis_index("c") * NS + lax.axis_index("s")
        accs = (a0, a1, a2, a3)
        off = jnp.where(wid < NBIG, wid * EBIG,
                        NBIG * EBIG + (wid - NBIG) * ESM)
        off = pl.multiple_of(off, 128)
        _zero_vmem(zc_hbm, accs)
        ones = jnp.ones((L,), jnp.float32)

        def work(sz):
            pltpu.sync_copy(edge_hbm.at[:, pl.ds(off, sz)],
                            ed_v.at[:, pl.ds(0, sz)])

            @plsc.parallel_loop(0, sz // L // 4)
            def _(j):
                for u in range(4):
                    d = ed_v[1, pl.ds((j * 4 + u) * L, L)]
                    plsc.addupdate_scatter(accs[u], [d], ones)

            pltpu.sync_copy(ed_v.at[0, pl.ds(0, sz)],
                            src_out.at[pl.ds(off, sz)])
            pltpu.sync_copy(ed_v.at[1, pl.ds(0, sz)],
                            dst_out.at[pl.ds(off, sz)])

        pl.when(wid < NBIG)(lambda: work(EBIG))
        pl.when(wid >= NBIG)(lambda: work(ESM))

        @plsc.parallel_loop(0, NP // (4 * L))
        def _(i):
            for u in range(4):
                s = pl.ds((i * 4 + u) * L, L)
                a0[s] = (a0[s] + a1[s]) + (a2[s] + a3[s])

        pltpu.sync_copy(a0, degp_out.at[wid])

    return k(edge_index, zc)


def _sc_scatter(g, src, dst, zc):
    @functools.partial(
        pl.kernel,
        out_type=jax.ShapeDtypeStruct((NW, OUT, NP), jnp.float32),
        mesh=_mesh(),
        compiler_params=_sc_params(),
        scratch_types=[
            pltpu.VMEM((NP,), jnp.float32),
            pltpu.VMEM((NP,), jnp.float32),
            pltpu.VMEM((EPW,), jnp.int32),
            pltpu.VMEM((EPW,), jnp.int32),
            pltpu.VMEM((NP,), jnp.float32),
            pltpu.VMEM((NP,), jnp.float32),
            pltpu.VMEM((NP,), jnp.float32),
            pltpu.VMEM((NP,), jnp.float32),
        ],
    )
    def k(g_hbm, src_hbm, dst_hbm, zc_hbm, out_hbm, g0_v, g1_v, src_v, dst_v,
          a0a, a1a, a0b, a1b):
        wid = lax.axis_index("c") * NS + lax.axis_index("s")
        pltpu.sync_copy(g_hbm.at[0], g0_v)
        pltpu.sync_copy(g_hbm.at[1], g1_v)
        pltpu.sync_copy(src_hbm.at[pl.ds(wid * EPW, EPW)], src_v)
        pltpu.sync_copy(dst_hbm.at[pl.ds(wid * EPW, EPW)], dst_v)
        _zero_vmem(zc_hbm, (a0a, a1a, a0b, a1b))
        pairs = ((a0a, a1a), (a0b, a1b))
        nu = (EPW // L) // 2

        @plsc.parallel_loop(0, nu)
        def _(j):
            for u in range(2):
                acc0, acc1 = pairs[u]
                s16 = src_v[pl.ds((j * 2 + u) * L, L)]
                d16 = dst_v[pl.ds((j * 2 + u) * L, L)]
                plsc.addupdate_scatter(acc0, [d16],
                                       plsc.load_gather(g0_v, [s16]))
                plsc.addupdate_scatter(acc1, [d16],
                                       plsc.load_gather(g1_v, [s16]))

        for j in range(nu * 2, EPW // L):
            s16 = src_v[pl.ds(j * L, L)]
            d16 = dst_v[pl.ds(j * L, L)]
            plsc.addupdate_scatter(a0a, [d16], plsc.load_gather(g0_v, [s16]))
            plsc.addupdate_scatter(a1a, [d16], plsc.load_gather(g1_v, [s16]))

        @plsc.parallel_loop(0, NP // (4 * L))
        def _(i):
            for u in range(4):
                s = pl.ds((i * 4 + u) * L, L)
                a0a[s] = a0a[s] + a0b[s]
                a1a[s] = a1a[s] + a1b[s]

        pltpu.sync_copy(a0a, out_hbm.at[wid, 0])
        pltpu.sync_copy(a1a, out_hbm.at[wid, 1])

    return k(g, src, dst, zc)


def _tc_matmul(in_feat, Wc):
    def k(x_ref, wc_ref, zt_ref):
        zt_ref[...] = lax.dot_general(wc_ref[...], x_ref[...],
                                      (((1,), (1,)), ((), ())),
                                      preferred_element_type=jnp.float32)

    return pl.pallas_call(
        k, out_shape=jax.ShapeDtypeStruct((8, N), jnp.float32),
    )(in_feat, Wc)


def _tc_prep(zt, degp):
    def k(zt_ref, degp_ref, meta_ref, g1_ref):
        zt = zt_ref[...]
        deg = jnp.sum(degp_ref[...], axis=0)[:N]
        n = lax.rsqrt(jnp.maximum(deg, 1.0))
        meta = jnp.concatenate(
            [zt[0:2] - zt[4:6],
             zt[2:4],
             n[None, :],
             jnp.zeros((3, N), jnp.float32)], axis=0)
        meta_ref[...] = jnp.pad(meta, ((0, 0), (0, NP - N)))
        g1 = zt[4:6] * n[None, :]
        g1_ref[...] = jnp.pad(g1, ((0, 0), (0, NP - N)))

    return pl.pallas_call(
        k,
        out_shape=(jax.ShapeDtypeStruct((8, NP), jnp.float32),
                   jax.ShapeDtypeStruct((OUT, NP), jnp.float32)),
    )(zt, degp)


def _tc_mid(meta, aggp1):
    def k(meta_ref, aggp_ref, g2_ref):
        agg1 = jnp.sum(aggp_ref[...], axis=0)
        y1 = meta_ref[2:4, :]
        n = meta_ref[4, :][None, :]
        g2_ref[...] = (-y1 + 2.0 * n * agg1) * n

    return pl.pallas_call(
        k, out_shape=jax.ShapeDtypeStruct((OUT, NP), jnp.float32),
    )(meta, aggp1)


def _tc_final(meta, aggp2):
    def k(meta_ref, aggp_ref, out_ref):
        agg2 = jnp.sum(aggp_ref[...], axis=0)
        base = meta_ref[0:2, :]
        n = meta_ref[4, :][None, :]
        out_ref[...] = (base + n * agg2)[:, :N]

    return pl.pallas_call(
        k, out_shape=jax.ShapeDtypeStruct((OUT, N), jnp.float32),
    )(meta, aggp2)


def kernel(in_feat, edge_index, W):
    W0 = W[:, 0:D]
    W1 = W[:, D:2 * D]
    W2 = W[:, 2 * D:3 * D]
    Wc = jnp.concatenate([W0, W1, W2, jnp.zeros((2, D), W.dtype)], axis=0)

    zc = jnp.zeros((NP,), jnp.float32)
    src, dst, degp = _sc_degree(edge_index, zc)
    zt = _tc_matmul(in_feat, Wc)
    meta, g1 = _tc_prep(zt, degp)
    aggp1 = _sc_scatter(g1, src, dst, zc)
    g2 = _tc_mid(meta, aggp1)
    aggp2 = _sc_scatter(g2, src, dst, zc)
    return _tc_final(meta, aggp2).T


# speedup vs baseline: 1.3255x; 1.3255x over previous
import functools

import jax
import jax.numpy as jnp
from jax import lax
from jax.experimental import pallas as pl
from jax.experimental.pallas import tpu as pltpu
from jax.experimental.pallas import tpu_sc as plsc

N = 10000
E = 320000
D = 128
OUT = 2
NC = 2
NS = 16
NW = NC * NS
L = 16
NP = 10240
EPW = E // NW

EBIG = 79 * 128
ESM = 78 * 128
NBIG = (E - NW * ESM) // (EBIG - ESM)

_SC_MESH = dict(core_axis_name="c", subcore_axis_name="s")


def _mesh():
    return plsc.VectorSubcoreMesh(num_cores=NC, num_subcores=NS, **_SC_MESH)


def _sc_params():
    return pltpu.CompilerParams(needs_layout_passes=False)


def _zero_vmem(refs, n):
    z = jnp.zeros((L,), jnp.float32)

    @plsc.parallel_loop(0, n // (8 * L))
    def _(i):
        for u in range(8):
            for r in refs:
                r[pl.ds((i * 8 + u) * L, L)] = z


def _sc_degree(edge_index):
    @functools.partial(
        pl.kernel,
        out_type=(jax.ShapeDtypeStruct((E,), jnp.int32),
                  jax.ShapeDtypeStruct((E,), jnp.int32),
                  jax.ShapeDtypeStruct((NW, NP), jnp.float32)),
        mesh=_mesh(),
        compiler_params=_sc_params(),
        scratch_types=[
            pltpu.VMEM((2, EBIG), jnp.int32),
            pltpu.VMEM((NP,), jnp.float32),
            pltpu.VMEM((NP,), jnp.float32),
            pltpu.VMEM((NP,), jnp.float32),
            pltpu.VMEM((NP,), jnp.float32),
        ],
    )
    def k(edge_hbm, src_out, dst_out, degp_out, ed_v, a0, a1, a2, a3):
        wid = lax.axis_index("c") * NS + lax.axis_index("s")
        accs = (a0, a1, a2, a3)
        off = jnp.where(wid < NBIG, wid * EBIG,
                        NBIG * EBIG + (wid - NBIG) * ESM)
        off = pl.multiple_of(off, 128)
        _zero_vmem(accs, NP)
        ones = jnp.ones((L,), jnp.float32)

        def work(sz):
            pltpu.sync_copy(edge_hbm.at[:, pl.ds(off, sz)],
                            ed_v.at[:, pl.ds(0, sz)])

            @plsc.parallel_loop(0, sz // L // 4)
            def _(j):
                for u in range(4):
                    d = ed_v[1, pl.ds((j * 4 + u) * L, L)]
                    plsc.addupdate_scatter(accs[u], [d], ones)

            pltpu.sync_copy(ed_v.at[0, pl.ds(0, sz)],
                            src_out.at[pl.ds(off, sz)])
            pltpu.sync_copy(ed_v.at[1, pl.ds(0, sz)],
                            dst_out.at[pl.ds(off, sz)])

        pl.when(wid < NBIG)(lambda: work(EBIG))
        pl.when(wid >= NBIG)(lambda: work(ESM))

        @plsc.parallel_loop(0, NP // (4 * L))
        def _(i):
            for u in range(4):
                s = pl.ds((i * 4 + u) * L, L)
                a0[s] = (a0[s] + a1[s]) + (a2[s] + a3[s])

        pltpu.sync_copy(a0, degp_out.at[wid])

    return k(edge_index)


def _sc_scatter(g, src, dst):
    @functools.partial(
        pl.kernel,
        out_type=jax.ShapeDtypeStruct((NW, OUT, NP), jnp.float32),
        mesh=_mesh(),
        compiler_params=_sc_params(),
        scratch_types=[
            pltpu.VMEM((NP,), jnp.float32),
            pltpu.VMEM((NP,), jnp.float32),
            pltpu.VMEM((EPW,), jnp.int32),
            pltpu.VMEM((EPW,), jnp.int32),
            pltpu.VMEM((NP,), jnp.float32),
            pltpu.VMEM((NP,), jnp.float32),
            pltpu.VMEM((NP,), jnp.float32),
            pltpu.VMEM((NP,), jnp.float32),
        ],
    )
    def k(g_hbm, src_hbm, dst_hbm, out_hbm, g0_v, g1_v, src_v, dst_v,
          a0a, a1a, a0b, a1b):
        wid = lax.axis_index("c") * NS + lax.axis_index("s")
        pltpu.sync_copy(g_hbm.at[0], g0_v)
        pltpu.sync_copy(g_hbm.at[1], g1_v)
        pltpu.sync_copy(src_hbm.at[pl.ds(wid * EPW, EPW)], src_v)
        pltpu.sync_copy(dst_hbm.at[pl.ds(wid * EPW, EPW)], dst_v)
        _zero_vmem((a0a, a1a, a0b, a1b), NP)
        pairs = ((a0a, a1a), (a0b, a1b))
        nu = (EPW // L) // 2

        @plsc.parallel_loop(0, nu)
        def _(j):
            for u in range(2):
                acc0, acc1 = pairs[u]
                s16 = src_v[pl.ds((j * 2 + u) * L, L)]
                d16 = dst_v[pl.ds((j * 2 + u) * L, L)]
                plsc.addupdate_scatter(acc0, [d16],
                                       plsc.load_gather(g0_v, [s16]))
                plsc.addupdate_scatter(acc1, [d16],
                                       plsc.load_gather(g1_v, [s16]))

        for j in range(nu * 2, EPW // L):
            s16 = src_v[pl.ds(j * L, L)]
            d16 = dst_v[pl.ds(j * L, L)]
            plsc.addupdate_scatter(a0a, [d16], plsc.load_gather(g0_v, [s16]))
            plsc.addupdate_scatter(a1a, [d16], plsc.load_gather(g1_v, [s16]))

        @plsc.parallel_loop(0, NP // (4 * L))
        def _(i):
            for u in range(4):
                s = pl.ds((i * 4 + u) * L, L)
                a0a[s] = a0a[s] + a0b[s]
                a1a[s] = a1a[s] + a1b[s]

        pltpu.sync_copy(a0a, out_hbm.at[wid, 0])
        pltpu.sync_copy(a1a, out_hbm.at[wid, 1])

    return k(g, src, dst)


def _tc_matmul(in_feat, Wc):
    def k(x_ref, wc_ref, zt_ref):
        zt_ref[...] = lax.dot_general(wc_ref[...], x_ref[...],
                                      (((1,), (1,)), ((), ())),
                                      preferred_element_type=jnp.float32)

    return pl.pallas_call(
        k, out_shape=jax.ShapeDtypeStruct((8, N), jnp.float32),
    )(in_feat, Wc)


def _tc_prep(zt, degp):
    def k(zt_ref, degp_ref, meta_ref, g1_ref):
        zt = zt_ref[...]
        deg = jnp.sum(degp_ref[...], axis=0)[:N]
        n = lax.rsqrt(jnp.maximum(deg, 1.0))
        meta = jnp.concatenate(
            [zt[0:2] - zt[4:6],
             zt[2:4],
             n[None, :],
             jnp.zeros((3, N), jnp.float32)], axis=0)
        meta_ref[...] = jnp.pad(meta, ((0, 0), (0, NP - N)))
        g1 = zt[4:6] * n[None, :]
        g1_ref[...] = jnp.pad(g1, ((0, 0), (0, NP - N)))

    return pl.pallas_call(
        k,
        out_shape=(jax.ShapeDtypeStruct((8, NP), jnp.float32),
                   jax.ShapeDtypeStruct((OUT, NP), jnp.float32)),
    )(zt, degp)


def _tc_mid(meta, aggp1):
    def k(meta_ref, aggp_ref, g2_ref):
        agg1 = jnp.sum(aggp_ref[...], axis=0)
        y1 = meta_ref[2:4, :]
        n = meta_ref[4, :][None, :]
        g2_ref[...] = (-y1 + 2.0 * n * agg1) * n

    return pl.pallas_call(
        k, out_shape=jax.ShapeDtypeStruct((OUT, NP), jnp.float32),
    )(meta, aggp1)


def _tc_final(meta, aggp2):
    def k(meta_ref, aggp_ref, out_ref):
        agg2 = jnp.sum(aggp_ref[...], axis=0)
        base = meta_ref[0:2, :]
        n = meta_ref[4, :][None, :]
        out_ref[...] = (base + n * agg2)[:, :N]

    return pl.pallas_call(
        k, out_shape=jax.ShapeDtypeStruct((OUT, N), jnp.float32),
    )(meta, aggp2)


def kernel(in_feat, edge_index, W):
    W0 = W[:, 0:D]
    W1 = W[:, D:2 * D]
    W2 = W[:, 2 * D:3 * D]
    Wc = jnp.concatenate([W0, W1, W2, jnp.zeros((2, D), W.dtype)], axis=0)

    src, dst, degp = _sc_degree(edge_index)
    zt = _tc_matmul(in_feat, Wc)
    meta, g1 = _tc_prep(zt, degp)
    aggp1 = _sc_scatter(g1, src, dst)
    g2 = _tc_mid(meta, aggp1)
    aggp2 = _sc_scatter(g2, src, dst)
    return _tc_final(meta, aggp2).T


# speedup vs baseline: 1.5721x; 1.1861x over previous
import functools

import jax
import jax.numpy as jnp
from jax import lax
from jax.experimental import pallas as pl
from jax.experimental.pallas import tpu as pltpu
from jax.experimental.pallas import tpu_sc as plsc

N = 10000
E = 320000
D = 128
OUT = 2
NC = 2
NS = 16
NW = NC * NS
L = 16
NP = 10240
EPW = E // NW

EBIG = 79 * 128
ESM = 78 * 128
NBIG = (E - NW * ESM) // (EBIG - ESM)

_SC_MESH = dict(core_axis_name="c", subcore_axis_name="s")


def _mesh():
    return plsc.VectorSubcoreMesh(num_cores=NC, num_subcores=NS, **_SC_MESH)


def _sc_params():
    return pltpu.CompilerParams(needs_layout_passes=False)


def _zero_vmem(refs, n):
    z = jnp.zeros((L,), jnp.float32)

    @plsc.parallel_loop(0, n // (8 * L))
    def _(i):
        for u in range(8):
            for r in refs:
                r[pl.ds((i * 8 + u) * L, L)] = z


def _sc_degree(edge_index):
    @functools.partial(
        pl.kernel,
        out_type=(jax.ShapeDtypeStruct((E,), jnp.int32),
                  jax.ShapeDtypeStruct((E,), jnp.int32),
                  jax.ShapeDtypeStruct((NW, NP), jnp.float32)),
        mesh=_mesh(),
        compiler_params=_sc_params(),
        scratch_types=[
            pltpu.VMEM((2, EBIG), jnp.int32),
            pltpu.VMEM((NP,), jnp.float32),
            pltpu.VMEM((NP,), jnp.float32),
            pltpu.VMEM((NP,), jnp.float32),
            pltpu.VMEM((NP,), jnp.float32),
            pltpu.SemaphoreType.DMA,
        ],
    )
    def k(edge_hbm, src_out, dst_out, degp_out, ed_v, a0, a1, a2, a3, sem):
        wid = lax.axis_index("c") * NS + lax.axis_index("s")
        accs = (a0, a1, a2, a3)
        off = jnp.where(wid < NBIG, wid * EBIG,
                        NBIG * EBIG + (wid - NBIG) * ESM)
        off = pl.multiple_of(off, 128)
        ones = jnp.ones((L,), jnp.float32)

        def work(sz):
            cp = pltpu.async_copy(edge_hbm.at[:, pl.ds(off, sz)],
                                  ed_v.at[:, pl.ds(0, sz)], sem)
            _zero_vmem(accs, NP)
            cp.wait()

            @plsc.parallel_loop(0, sz // L // 4)
            def _(j):
                for u in range(4):
                    d = ed_v[1, pl.ds((j * 4 + u) * L, L)]
                    plsc.addupdate_scatter(accs[u], [d], ones)

            pltpu.sync_copy(ed_v.at[0, pl.ds(0, sz)],
                            src_out.at[pl.ds(off, sz)])
            pltpu.sync_copy(ed_v.at[1, pl.ds(0, sz)],
                            dst_out.at[pl.ds(off, sz)])

        pl.when(wid < NBIG)(lambda: work(EBIG))
        pl.when(wid >= NBIG)(lambda: work(ESM))

        @plsc.parallel_loop(0, NP // (4 * L))
        def _(i):
            for u in range(4):
                s = pl.ds((i * 4 + u) * L, L)
                a0[s] = (a0[s] + a1[s]) + (a2[s] + a3[s])

        pltpu.sync_copy(a0, degp_out.at[wid])

    return k(edge_index)


def _sc_scatter(g, src, dst):
    @functools.partial(
        pl.kernel,
        out_type=jax.ShapeDtypeStruct((NW, OUT, NP), jnp.float32),
        mesh=_mesh(),
        compiler_params=_sc_params(),
        scratch_types=[
            pltpu.VMEM((NP,), jnp.float32),
            pltpu.VMEM((NP,), jnp.float32),
            pltpu.VMEM((EPW,), jnp.int32),
            pltpu.VMEM((EPW,), jnp.int32),
            pltpu.VMEM((NP,), jnp.float32),
            pltpu.VMEM((NP,), jnp.float32),
            pltpu.VMEM((NP,), jnp.float32),
            pltpu.VMEM((NP,), jnp.float32),
            pltpu.SemaphoreType.DMA,
        ],
    )
    def k(g_hbm, src_hbm, dst_hbm, out_hbm, g0_v, g1_v, src_v, dst_v,
          a0a, a1a, a0b, a1b, sem):
        wid = lax.axis_index("c") * NS + lax.axis_index("s")
        cps = [pltpu.async_copy(g_hbm.at[0], g0_v, sem),
               pltpu.async_copy(g_hbm.at[1], g1_v, sem),
               pltpu.async_copy(src_hbm.at[pl.ds(wid * EPW, EPW)], src_v, sem),
               pltpu.async_copy(dst_hbm.at[pl.ds(wid * EPW, EPW)], dst_v, sem)]
        _zero_vmem((a0a, a1a, a0b, a1b), NP)
        for cp in cps:
            cp.wait()
        pairs = ((a0a, a1a), (a0b, a1b))
        nu = (EPW // L) // 2

        @plsc.parallel_loop(0, nu)
        def _(j):
            for u in range(2):
                acc0, acc1 = pairs[u]
                s16 = src_v[pl.ds((j * 2 + u) * L, L)]
                d16 = dst_v[pl.ds((j * 2 + u) * L, L)]
                plsc.addupdate_scatter(acc0, [d16],
                                       plsc.load_gather(g0_v, [s16]))
                plsc.addupdate_scatter(acc1, [d16],
                                       plsc.load_gather(g1_v, [s16]))

        for j in range(nu * 2, EPW // L):
            s16 = src_v[pl.ds(j * L, L)]
            d16 = dst_v[pl.ds(j * L, L)]
            plsc.addupdate_scatter(a0a, [d16], plsc.load_gather(g0_v, [s16]))
            plsc.addupdate_scatter(a1a, [d16], plsc.load_gather(g1_v, [s16]))

        @plsc.parallel_loop(0, NP // (4 * L))
        def _(i):
            for u in range(4):
                s = pl.ds((i * 4 + u) * L, L)
                a0a[s] = a0a[s] + a0b[s]
                a1a[s] = a1a[s] + a1b[s]

        pltpu.sync_copy(a0a, out_hbm.at[wid, 0])
        pltpu.sync_copy(a1a, out_hbm.at[wid, 1])

    return k(g, src, dst)


def _tc_matmul(in_feat, Wc):
    def k(x_ref, wc_ref, zt_ref):
        zt_ref[...] = lax.dot_general(wc_ref[...], x_ref[...],
                                      (((1,), (1,)), ((), ())),
                                      preferred_element_type=jnp.float32)

    return pl.pallas_call(
        k, out_shape=jax.ShapeDtypeStruct((8, N), jnp.float32),
    )(in_feat, Wc)


def _tc_prep(zt, degp):
    def k(zt_ref, degp_ref, meta_ref, g1_ref):
        zt = zt_ref[...]
        deg = jnp.sum(degp_ref[...], axis=0)[:N]
        n = lax.rsqrt(jnp.maximum(deg, 1.0))
        meta = jnp.concatenate(
            [zt[0:2] - zt[4:6],
             zt[2:4],
             n[None, :],
             jnp.zeros((3, N), jnp.float32)], axis=0)
        meta_ref[...] = jnp.pad(meta, ((0, 0), (0, NP - N)))
        g1 = zt[4:6] * n[None, :]
        g1_ref[...] = jnp.pad(g1, ((0, 0), (0, NP - N)))

    return pl.pallas_call(
        k,
        out_shape=(jax.ShapeDtypeStruct((8, NP), jnp.float32),
                   jax.ShapeDtypeStruct((OUT, NP), jnp.float32)),
    )(zt, degp)


def _tc_mid(meta, aggp1):
    def k(meta_ref, aggp_ref, g2_ref):
        agg1 = jnp.sum(aggp_ref[...], axis=0)
        y1 = meta_ref[2:4, :]
        n = meta_ref[4, :][None, :]
        g2_ref[...] = (-y1 + 2.0 * n * agg1) * n

    return pl.pallas_call(
        k, out_shape=jax.ShapeDtypeStruct((OUT, NP), jnp.float32),
    )(meta, aggp1)


def _tc_final(meta, aggp2):
    def k(meta_ref, aggp_ref, out_ref):
        agg2 = jnp.sum(aggp_ref[...], axis=0)
        base = meta_ref[0:2, :]
        n = meta_ref[4, :][None, :]
        out_ref[...] = (base + n * agg2)[:, :N]

    return pl.pallas_call(
        k, out_shape=jax.ShapeDtypeStruct((OUT, N), jnp.float32),
    )(meta, aggp2)


def kernel(in_feat, edge_index, W):
    W0 = W[:, 0:D]
    W1 = W[:, D:2 * D]
    W2 = W[:, 2 * D:3 * D]
    Wc = jnp.concatenate([W0, W1, W2, jnp.zeros((2, D), W.dtype)], axis=0)

    src, dst, degp = _sc_degree(edge_index)
    zt = _tc_matmul(in_feat, Wc)
    meta, g1 = _tc_prep(zt, degp)
    aggp1 = _sc_scatter(g1, src, dst)
    g2 = _tc_mid(meta, aggp1)
    aggp2 = _sc_scatter(g2, src, dst)
    return _tc_final(meta, aggp2).T


# speedup vs baseline: 1.5730x; 1.0005x over previous
import functools

import jax
import jax.numpy as jnp
from jax import lax
from jax.experimental import pallas as pl
from jax.experimental.pallas import tpu as pltpu
from jax.experimental.pallas import tpu_sc as plsc

N = 10000
E = 320000
D = 128
OUT = 2
NC = 2
NS = 16
NW = NC * NS
L = 16
NP = 10240
EPW = E // NW

EBIG = 79 * 128
ESM = 78 * 128
NBIG = (E - NW * ESM) // (EBIG - ESM)

_SC_MESH = dict(core_axis_name="c", subcore_axis_name="s")


def _mesh():
    return plsc.VectorSubcoreMesh(num_cores=NC, num_subcores=NS, **_SC_MESH)


def _sc_params():
    return pltpu.CompilerParams(needs_layout_passes=False)


def _zero_vmem(refs, n):
    z = jnp.zeros((L,), jnp.float32)

    @plsc.parallel_loop(0, n // (8 * L))
    def _(i):
        for u in range(8):
            for r in refs:
                r[pl.ds((i * 8 + u) * L, L)] = z


def _sc_degree(edge_index):
    @functools.partial(
        pl.kernel,
        out_type=(jax.ShapeDtypeStruct((E,), jnp.int32),
                  jax.ShapeDtypeStruct((E,), jnp.int32),
                  jax.ShapeDtypeStruct((NW, NP), jnp.float32)),
        mesh=_mesh(),
        compiler_params=_sc_params(),
        scratch_types=[
            pltpu.VMEM((2, EBIG), jnp.int32),
            pltpu.VMEM((NP,), jnp.float32),
            pltpu.VMEM((NP,), jnp.float32),
            pltpu.VMEM((NP,), jnp.float32),
            pltpu.VMEM((NP,), jnp.float32),
            pltpu.SemaphoreType.DMA,
        ],
    )
    def k(edge_hbm, src_out, dst_out, degp_out, ed_v, a0, a1, a2, a3, sem):
        wid = lax.axis_index("c") * NS + lax.axis_index("s")
        accs = (a0, a1, a2, a3)
        off = jnp.where(wid < NBIG, wid * EBIG,
                        NBIG * EBIG + (wid - NBIG) * ESM)
        off = pl.multiple_of(off, 128)
        ones = jnp.ones((L,), jnp.float32)

        def work(sz):
            cp = pltpu.async_copy(edge_hbm.at[:, pl.ds(off, sz)],
                                  ed_v.at[:, pl.ds(0, sz)], sem)
            _zero_vmem(accs, NP)
            cp.wait()

            @plsc.parallel_loop(0, sz // L // 4)
            def _(j):
                for u in range(4):
                    d = ed_v[1, pl.ds((j * 4 + u) * L, L)]
                    plsc.addupdate_scatter(accs[u], [d], ones)

            cp0 = pltpu.async_copy(ed_v.at[0, pl.ds(0, sz)],
                                   src_out.at[pl.ds(off, sz)], sem)
            cp1 = pltpu.async_copy(ed_v.at[1, pl.ds(0, sz)],
                                   dst_out.at[pl.ds(off, sz)], sem)

            @plsc.parallel_loop(0, NP // (4 * L))
            def _(i):
                for u in range(4):
                    sl = pl.ds((i * 4 + u) * L, L)
                    a0[sl] = (a0[sl] + a1[sl]) + (a2[sl] + a3[sl])

            cp0.wait()
            cp1.wait()

        pl.when(wid < NBIG)(lambda: work(EBIG))
        pl.when(wid >= NBIG)(lambda: work(ESM))

        pltpu.sync_copy(a0, degp_out.at[wid])

    return k(edge_index)


def _sc_scatter(g, src, dst):
    @functools.partial(
        pl.kernel,
        out_type=jax.ShapeDtypeStruct((NW, OUT, NP), jnp.float32),
        mesh=_mesh(),
        compiler_params=_sc_params(),
        scratch_types=[
            pltpu.VMEM((NP,), jnp.float32),
            pltpu.VMEM((NP,), jnp.float32),
            pltpu.VMEM((EPW,), jnp.int32),
            pltpu.VMEM((EPW,), jnp.int32),
            pltpu.VMEM((NP,), jnp.float32),
            pltpu.VMEM((NP,), jnp.float32),
            pltpu.VMEM((NP,), jnp.float32),
            pltpu.VMEM((NP,), jnp.float32),
            pltpu.SemaphoreType.DMA,
        ],
    )
    def k(g_hbm, src_hbm, dst_hbm, out_hbm, g0_v, g1_v, src_v, dst_v,
          a0a, a1a, a0b, a1b, sem):
        wid = lax.axis_index("c") * NS + lax.axis_index("s")
        cps = [pltpu.async_copy(g_hbm.at[0], g0_v, sem),
               pltpu.async_copy(g_hbm.at[1], g1_v, sem),
               pltpu.async_copy(src_hbm.at[pl.ds(wid * EPW, EPW)], src_v, sem),
               pltpu.async_copy(dst_hbm.at[pl.ds(wid * EPW, EPW)], dst_v, sem)]
        _zero_vmem((a0a, a1a, a0b, a1b), NP)
        for cp in cps:
            cp.wait()
        pairs = ((a0a, a1a), (a0b, a1b))
        nu = (EPW // L) // 2

        @plsc.parallel_loop(0, nu)
        def _(j):
            for u in range(2):
                acc0, acc1 = pairs[u]
                s16 = src_v[pl.ds((j * 2 + u) * L, L)]
                d16 = dst_v[pl.ds((j * 2 + u) * L, L)]
                plsc.addupdate_scatter(acc0, [d16],
                                       plsc.load_gather(g0_v, [s16]))
                plsc.addupdate_scatter(acc1, [d16],
                                       plsc.load_gather(g1_v, [s16]))

        for j in range(nu * 2, EPW // L):
            s16 = src_v[pl.ds(j * L, L)]
            d16 = dst_v[pl.ds(j * L, L)]
            plsc.addupdate_scatter(a0a, [d16], plsc.load_gather(g0_v, [s16]))
            plsc.addupdate_scatter(a1a, [d16], plsc.load_gather(g1_v, [s16]))

        @plsc.parallel_loop(0, NP // (4 * L))
        def _(i):
            for u in range(4):
                s = pl.ds((i * 4 + u) * L, L)
                a0a[s] = a0a[s] + a0b[s]
                a1a[s] = a1a[s] + a1b[s]

        pltpu.sync_copy(a0a, out_hbm.at[wid, 0])
        pltpu.sync_copy(a1a, out_hbm.at[wid, 1])

    return k(g, src, dst)


def _tc_matmul(in_feat, Wc):
    def k(x_ref, wc_ref, zt_ref):
        zt_ref[...] = lax.dot_general(wc_ref[...], x_ref[...],
                                      (((1,), (1,)), ((), ())),
                                      preferred_element_type=jnp.float32)

    return pl.pallas_call(
        k, out_shape=jax.ShapeDtypeStruct((8, N), jnp.float32),
    )(in_feat, Wc)


def _tc_prep(zt, degp):
    def k(zt_ref, degp_ref, meta_ref, g1_ref):
        zt = zt_ref[...]
        deg = jnp.sum(degp_ref[...], axis=0)[:N]
        n = lax.rsqrt(jnp.maximum(deg, 1.0))
        meta = jnp.concatenate(
            [zt[0:2] - zt[4:6],
             zt[2:4],
             n[None, :],
             jnp.zeros((3, N), jnp.float32)], axis=0)
        meta_ref[...] = jnp.pad(meta, ((0, 0), (0, NP - N)))
        g1 = zt[4:6] * n[None, :]
        g1_ref[...] = jnp.pad(g1, ((0, 0), (0, NP - N)))

    return pl.pallas_call(
        k,
        out_shape=(jax.ShapeDtypeStruct((8, NP), jnp.float32),
                   jax.ShapeDtypeStruct((OUT, NP), jnp.float32)),
    )(zt, degp)


def _tc_mid(meta, aggp1):
    def k(meta_ref, aggp_ref, g2_ref):
        agg1 = jnp.sum(aggp_ref[...], axis=0)
        y1 = meta_ref[2:4, :]
        n = meta_ref[4, :][None, :]
        g2_ref[...] = (-y1 + 2.0 * n * agg1) * n

    return pl.pallas_call(
        k, out_shape=jax.ShapeDtypeStruct((OUT, NP), jnp.float32),
    )(meta, aggp1)


def _tc_final(meta, aggp2):
    def k(meta_ref, aggp_ref, out_ref):
        agg2 = jnp.sum(aggp_ref[...], axis=0)
        base = meta_ref[0:2, :]
        n = meta_ref[4, :][None, :]
        out_ref[...] = (base + n * agg2)[:, :N]

    return pl.pallas_call(
        k, out_shape=jax.ShapeDtypeStruct((OUT, N), jnp.float32),
    )(meta, aggp2)


def kernel(in_feat, edge_index, W):
    W0 = W[:, 0:D]
    W1 = W[:, D:2 * D]
    W2 = W[:, 2 * D:3 * D]
    Wc = jnp.concatenate([W0, W1, W2, jnp.zeros((2, D), W.dtype)], axis=0)

    src, dst, degp = _sc_degree(edge_index)
    zt = _tc_matmul(in_feat, Wc)
    meta, g1 = _tc_prep(zt, degp)
    aggp1 = _sc_scatter(g1, src, dst)
    g2 = _tc_mid(meta, aggp1)
    aggp2 = _sc_scatter(g2, src, dst)
    return _tc_final(meta, aggp2).T
